# baseline (speedup 1.0000x reference)
import functools

import jax
import jax.numpy as jnp
from jax import lax
from jax.experimental import pallas as pl
from jax.experimental.pallas import tpu as pltpu
from jax.experimental.pallas import tpu_sc as plsc

NEG = 0.2
EPS = 1e-16
CHUNK = 96
NW = 32
DR = 80
NB = 3


def _leaky_exp(e):
    return jnp.exp(jnp.maximum(e, NEG * e))


def _tc1_body(x_ref, w_ref, as_ref, ad_ref, s_ref, d_ref):
    d1 = w_ref.shape[1]
    h1 = as_ref.shape[1]
    h = jnp.dot(x_ref[...], w_ref[...], preferred_element_type=jnp.float32)
    a_s = jnp.dot(h, as_ref[...], preferred_element_type=jnp.float32)
    a_d = jnp.dot(h, ad_ref[...], preferred_element_type=jnp.float32)
    s_ref[:, 0:d1] = h
    s_ref[:, d1:d1 + h1] = a_s
    s_ref[:, d1 + h1:d1 + 2 * h1] = jnp.zeros_like(a_s)
    d_ref[:, 0:h1] = a_d
    d_ref[:, h1:2 * h1] = jnp.zeros_like(a_d)


def _tc2_body(s1_ref, d1_ref, a0_ref, a1_ref, b1_ref, w2_ref, v2s_ref,
              v2d_ref, s2_ref, d2_ref):
    d1 = w2_ref.shape[0]
    h1 = d1_ref.shape[1] // 2
    c1 = d1 // h1
    n2 = w2_ref.shape[1]
    h = s1_ref[:, 0:d1]
    exs = _leaky_exp(s1_ref[:, d1:d1 + h1] + d1_ref[:, 0:h1])
    num = a0_ref[:, 0:d1] + a1_ref[:, 0:d1]
    den = a0_ref[:, d1:d1 + h1] + a1_ref[:, d1:d1 + h1] + exs
    r = (lax.broadcasted_iota(jnp.int32, (h1, d1), 1) // c1
         == lax.broadcasted_iota(jnp.int32, (h1, d1), 0)).astype(jnp.float32)
    num = num + jnp.dot(exs, r, preferred_element_type=jnp.float32) * h
    den64 = jnp.dot(den, r, preferred_element_type=jnp.float32)
    out1 = num / (den64 + EPS) + b1_ref[...]
    hh = jnp.where(out1 > 0, out1, jnp.exp(out1) - 1.0)
    h2 = jnp.dot(hh, w2_ref[...], preferred_element_type=jnp.float32)
    a2s = jnp.sum(h2 * v2s_ref[...], axis=1, keepdims=True)
    a2d = jnp.sum(h2 * v2d_ref[...], axis=1, keepdims=True)
    s2_ref[:, 0:n2] = h2
    s2_ref[:, n2:2 * n2] = jnp.broadcast_to(a2s, h2.shape)
    d2_ref[...] = jnp.broadcast_to(a2d, h2.shape)


def _tc3_body(s2_ref, d2_ref, a0_ref, a1_ref, b2_ref, o_ref):
    n2 = o_ref.shape[1]
    h2 = s2_ref[:, 0:n2]
    exs = _leaky_exp(s2_ref[:, n2:2 * n2] + d2_ref[...])
    num = a0_ref[:, 0:n2] + a1_ref[:, 0:n2] + exs * h2
    den = a0_ref[:, n2:2 * n2] + a1_ref[:, n2:2 * n2] + exs
    out = num / (den + EPS) + b2_ref[...]
    m = jnp.max(out, axis=1, keepdims=True)
    lse = m + jnp.log(jnp.sum(jnp.exp(out - m), axis=1, keepdims=True))
    o_ref[...] = out - lse


def _make_sc_pass(n_pad, sw, dw, nct, compute_chunk):
    rows_sub = n_pad // 16
    hops = rows_sub // DR
    mesh = plsc.VectorSubcoreMesh(core_axis_name="c", subcore_axis_name="s",
                                  num_cores=2, num_subcores=16)

    nb = NB
    full = nct // nb
    rem = nct - nb * full

    @functools.partial(
        pl.kernel,
        out_type=jax.ShapeDtypeStruct((2 * n_pad, sw), jnp.float32),
        mesh=mesh,
        scratch_types=[
            pltpu.VMEM_SHARED((n_pad, sw), jnp.float32),
            pltpu.VMEM((nct, CHUNK), jnp.int32),
            pltpu.VMEM((nct, CHUNK), jnp.int32),
            pltpu.VMEM((nb, CHUNK, sw), jnp.float32),
            pltpu.VMEM((nb, CHUNK, dw), jnp.float32),
            pltpu.VMEM((nb, CHUNK, sw), jnp.float32),
            pltpu.SemaphoreType.DMA,
            [pltpu.SemaphoreType.DMA] * nb,
            [pltpu.SemaphoreType.DMA] * nb,
            [pltpu.SemaphoreType.DMA] * nb,
        ],
        compiler_params=pltpu.CompilerParams(use_tc_tiling_on_sc=False),
    )
    def sc_pass(s_hbm, d_hbm, src_hbm, dst_hbm, acc_hbm, acc_sh, src_all,
                dst_all, srows, drows, outb, sem_i, sem_g, sem_d, sem_w):
        c = lax.axis_index("c")
        s = lax.axis_index("s")
        wid = c * 16 + s

        gi1 = pltpu.async_copy(src_hbm.at[pl.ds(wid * nct, nct)], src_all,
                               sem_i)
        gi2 = pltpu.async_copy(dst_hbm.at[pl.ds(wid * nct, nct)], dst_all,
                               sem_i)

        def zrow(j, _):
            def zcol(k, __):
                outb[0, j, pl.ds(k * 16, 16)] = jnp.zeros((16,), jnp.float32)
                return 0
            return lax.fori_loop(0, sw // 16, zcol, 0)
        lax.fori_loop(0, CHUNK, zrow, 0)

        def zhop(t, _):
            pltpu.sync_copy(outb.at[0, pl.ds(0, DR)],
                            acc_sh.at[pl.ds(s * rows_sub + t * DR, DR)])
            return 0
        lax.fori_loop(0, hops, zhop, 0)
        gi1.wait()
        gi2.wait()

        def gathers(i, b):
            return (pltpu.make_async_copy(s_hbm.at[src_all.at[i]],
                                          srows.at[b], sem_g[b]),
                    pltpu.make_async_copy(d_hbm.at[dst_all.at[i]],
                                          drows.at[b], sem_d[b]))

        def scatter(i, b):
            return pltpu.make_async_copy(outb.at[b], acc_sh.at[dst_all.at[i]],
                                         sem_w[b])

        for b in range(nb):
            g1, g2 = gathers(b, b)
            g1.start()
            g2.start()
        plsc.subcore_barrier()

        def stage(i, b, t):
            g1, g2 = gathers(i, b)
            g1.wait()
            g2.wait()

            @pl.when(t > 0)
            def _():
                scatter(i, b).wait()
            compute_chunk(srows.at[b], drows.at[b], outb.at[b])
            scatter(i, b).start(add=True)

            @pl.when(i + nb < nct)
            def _():
                p1, p2 = gathers(i + nb, b)
                p1.start()
                p2.start()

        def round_(t, _):
            for b in range(nb):
                stage(nb * t + b, b, t)
            return 0
        lax.fori_loop(0, full, round_, 0)
        for b in range(rem):
            stage(nb * full + b, b, full)
        for b in range(nb):
            scatter(b, b).wait()
        plsc.subcore_barrier()

        def drain_hop(t, b):
            r0 = s * rows_sub + t * DR

            @pl.when(t >= nb)
            def _():
                pltpu.make_async_copy(outb.at[b, pl.ds(0, DR)],
                                      acc_hbm.at[pl.ds(0, DR)],
                                      sem_w[b]).wait()
            pltpu.sync_copy(acc_sh.at[pl.ds(r0, DR)], outb.at[b, pl.ds(0, DR)])
            pltpu.async_copy(outb.at[b, pl.ds(0, DR)],
                             acc_hbm.at[pl.ds(c * n_pad + r0, DR)],
                             sem_w[b])

        def dround(t, _):
            for b in range(nb):
                drain_hop(nb * t + b, b)
            return 0
        lax.fori_loop(0, hops // nb, dround, 0)
        for b in range(hops - nb * (hops // nb)):
            drain_hop(nb * (hops // nb) + b, b)
        for b in range(min(nb, hops)):
            pltpu.make_async_copy(outb.at[b, pl.ds(0, DR)],
                                  acc_hbm.at[pl.ds(0, DR)],
                                  sem_w[b]).wait()

    return sc_pass


def _cc_layer1(d1):
    def compute_chunk(srows, drows, outb):
        half = lax.shift_right_logical(lax.iota(jnp.int32, 16), 3)

        @plsc.parallel_loop(0, CHUNK, unroll=8)
        def body(j):
            ea = srows[j, pl.ds(d1, 16)] + drows[j, pl.ds(0, 16)]
            ex = _leaky_exp(ea)
            for v in range(d1 // 16):
                idx = half + 2 * v
                m = ex.at[idx].get(mode="promise_in_bounds")
                outb[j, pl.ds(16 * v, 16)] = srows[j, pl.ds(16 * v, 16)] * m
            outb[j, pl.ds(d1, 16)] = ex
    return compute_chunk


def _cc_layer2(n2):
    def compute_chunk(srows, drows, outb):
        @plsc.parallel_loop(0, CHUNK, unroll=8)
        def body(j):
            ea = srows[j, pl.ds(n2, 16)] + drows[j, pl.ds(0, 16)]
            ex = _leaky_exp(ea)
            outb[j, pl.ds(0, 16)] = srows[j, pl.ds(0, 16)] * ex
            outb[j, pl.ds(16, 16)] = ex
    return compute_chunk


def kernel(x, edge_index, W1, att_src1, att_dst1, b1, W2, att_src2, att_dst2,
           b2):
    N, F = x.shape
    H1, C1 = att_src1.shape[1], att_src1.shape[2]
    D1 = H1 * C1
    n_cls = W2.shape[1]
    E = edge_index.shape[1]

    n_pad = -(-N // 2048) * 2048
    nct = NB * -(-E // (NW * CHUNK * NB))
    e_pad = nct * NW * CHUNK
    sw1, dw1 = D1 + 16, 16
    sw2, dw2 = 2 * n_cls, n_cls

    x_pad = jnp.zeros((n_pad, F), jnp.float32).at[:N].set(x)
    fill = N + jnp.arange(e_pad - E, dtype=jnp.int32) % (n_pad - N)
    srcp = (jnp.concatenate([edge_index[0].astype(jnp.int32), fill])
            .reshape(NW * nct, CHUNK))
    dstp = (jnp.concatenate([edge_index[1].astype(jnp.int32), fill])
            .reshape(NW * nct, CHUNK))

    eye = jnp.eye(H1, dtype=jnp.float32)
    A_s = (att_src1[0][:, :, None] * eye[:, None, :]).reshape(D1, H1)
    A_d = (att_dst1[0][:, :, None] * eye[:, None, :]).reshape(D1, H1)
    v2s = att_src2.reshape(1, n_cls)
    v2d = att_dst2.reshape(1, n_cls)
    b1r = b1.reshape(1, D1)
    b2r = b2.reshape(1, n_cls)

    B = 1024
    grid = n_pad // B

    S1, T1 = pl.pallas_call(
        _tc1_body,
        grid=(grid,),
        in_specs=[
            pl.BlockSpec((B, F), lambda i: (i, 0)),
            pl.BlockSpec((F, D1), lambda i: (0, 0)),
            pl.BlockSpec((D1, H1), lambda i: (0, 0)),
            pl.BlockSpec((D1, H1), lambda i: (0, 0)),
        ],
        out_specs=[
            pl.BlockSpec((B, sw1), lambda i: (i, 0)),
            pl.BlockSpec((B, dw1), lambda i: (i, 0)),
        ],
        out_shape=[
            jax.ShapeDtypeStruct((n_pad, sw1), jnp.float32),
            jax.ShapeDtypeStruct((n_pad, dw1), jnp.float32),
        ],
    )(x_pad, W1, A_s, A_d)

    acc1 = _make_sc_pass(n_pad, sw1, dw1, nct,
                         _cc_layer1(D1))(S1, T1, srcp, dstp)

    S2, T2 = pl.pallas_call(
        _tc2_body,
        grid=(grid,),
        in_specs=[
            pl.BlockSpec((B, sw1), lambda i: (i, 0)),
            pl.BlockSpec((B, dw1), lambda i: (i, 0)),
            pl.BlockSpec((B, sw1), lambda i: (i, 0)),
            pl.BlockSpec((B, sw1), lambda i, g=grid: (i + g, 0)),
            pl.BlockSpec((1, D1), lambda i: (0, 0)),
            pl.BlockSpec((D1, n_cls), lambda i: (0, 0)),
            pl.BlockSpec((1, n_cls), lambda i: (0, 0)),
            pl.BlockSpec((1, n_cls), lambda i: (0, 0)),
        ],
        out_specs=[
            pl.BlockSpec((B, sw2), lambda i: (i, 0)),
            pl.BlockSpec((B, dw2), lambda i: (i, 0)),
        ],
        out_shape=[
            jax.ShapeDtypeStruct((n_pad, sw2), jnp.float32),
            jax.ShapeDtypeStruct((n_pad, dw2), jnp.float32),
        ],
    )(S1, T1, acc1, acc1, b1r, W2, v2s, v2d)

    acc2 = _make_sc_pass(n_pad, sw2, dw2, nct,
                         _cc_layer2(n_cls))(S2, T2, srcp, dstp)

    out = pl.pallas_call(
        _tc3_body,
        grid=(grid,),
        in_specs=[
            pl.BlockSpec((B, sw2), lambda i: (i, 0)),
            pl.BlockSpec((B, dw2), lambda i: (i, 0)),
            pl.BlockSpec((B, sw2), lambda i: (i, 0)),
            pl.BlockSpec((B, sw2), lambda i, g=grid: (i + g, 0)),
            pl.BlockSpec((1, n_cls), lambda i: (0, 0)),
        ],
        out_specs=pl.BlockSpec((B, n_cls), lambda i: (i, 0)),
        out_shape=jax.ShapeDtypeStruct((n_pad, n_cls), jnp.float32),
    )(S2, T2, acc2, acc2, b2r)

    return out[:N]


# speedup vs baseline: 1.0747x; 1.0747x over previous
import functools

import jax
import jax.numpy as jnp
from jax import lax
from jax.experimental import pallas as pl
from jax.experimental.pallas import tpu as pltpu
from jax.experimental.pallas import tpu_sc as plsc

NEG = 0.2
EPS = 1e-16
CHUNK = 96
NW = 32
DR = 80
NB = 3


def _leaky_exp(e):
    return jnp.exp(jnp.maximum(e, NEG * e))


def _tc1_body(x_ref, w_ref, as_ref, ad_ref, s_ref, d_ref):
    d1 = w_ref.shape[1]
    h1 = as_ref.shape[1]
    h = jnp.dot(x_ref[...], w_ref[...], preferred_element_type=jnp.float32)
    a_s = jnp.dot(h, as_ref[...], preferred_element_type=jnp.float32)
    a_d = jnp.dot(h, ad_ref[...], preferred_element_type=jnp.float32)
    s_ref[:, 0:d1] = h
    s_ref[:, d1:d1 + h1] = a_s
    s_ref[:, d1 + h1:d1 + 2 * h1] = jnp.zeros_like(a_s)
    d_ref[:, 0:h1] = a_d
    d_ref[:, h1:2 * h1] = jnp.zeros_like(a_d)


def _tc2_body(s1_ref, d1_ref, a0_ref, a1_ref, b1_ref, w2_ref, v2s_ref,
              v2d_ref, s2_ref, d2_ref):
    d1 = w2_ref.shape[0]
    h1 = d1_ref.shape[1] // 2
    c1 = d1 // h1
    n2 = w2_ref.shape[1]
    h = s1_ref[:, 0:d1]
    exs = _leaky_exp(s1_ref[:, d1:d1 + h1] + d1_ref[:, 0:h1])
    num = a0_ref[:, 0:d1] + a1_ref[:, 0:d1]
    den = a0_ref[:, d1:d1 + h1] + a1_ref[:, d1:d1 + h1] + exs
    r = (lax.broadcasted_iota(jnp.int32, (h1, d1), 1) // c1
         == lax.broadcasted_iota(jnp.int32, (h1, d1), 0)).astype(jnp.float32)
    num = num + jnp.dot(exs, r, preferred_element_type=jnp.float32) * h
    den64 = jnp.dot(den, r, preferred_element_type=jnp.float32)
    out1 = num / (den64 + EPS) + b1_ref[...]
    hh = jnp.where(out1 > 0, out1, jnp.exp(out1) - 1.0)
    h2 = jnp.dot(hh, w2_ref[...], preferred_element_type=jnp.float32)
    a2s = jnp.sum(h2 * v2s_ref[...], axis=1, keepdims=True)
    a2d = jnp.sum(h2 * v2d_ref[...], axis=1, keepdims=True)
    s2_ref[:, 0:n2] = h2
    s2_ref[:, n2:2 * n2] = jnp.broadcast_to(a2s, h2.shape)
    d2_ref[...] = jnp.broadcast_to(a2d, h2.shape)


def _tc3_body(s2_ref, d2_ref, a0_ref, a1_ref, b2_ref, o_ref):
    n2 = o_ref.shape[1]
    h2 = s2_ref[:, 0:n2]
    exs = _leaky_exp(s2_ref[:, n2:2 * n2] + d2_ref[...])
    num = a0_ref[:, 0:n2] + a1_ref[:, 0:n2] + exs * h2
    den = a0_ref[:, n2:2 * n2] + a1_ref[:, n2:2 * n2] + exs
    out = num / (den + EPS) + b2_ref[...]
    m = jnp.max(out, axis=1, keepdims=True)
    lse = m + jnp.log(jnp.sum(jnp.exp(out - m), axis=1, keepdims=True))
    o_ref[...] = out - lse


def _make_sc_pass(n_pad, sw, dw, nct, n0, n_real, compute_chunk):
    rows_sub = n_pad // 16
    hops = rows_sub // DR
    mesh = plsc.VectorSubcoreMesh(core_axis_name="c", subcore_axis_name="s",
                                  num_cores=2, num_subcores=16)

    s0 = nct * CHUNK
    nfill = s0 - n_real
    nb = NB
    full = nct // nb
    rem = nct - nb * full

    @functools.partial(
        pl.kernel,
        out_type=jax.ShapeDtypeStruct((2 * n_pad, sw), jnp.float32),
        mesh=mesh,
        scratch_types=[
            pltpu.VMEM_SHARED((n_pad, sw), jnp.float32),
            pltpu.VMEM((s0,), jnp.int32),
            pltpu.VMEM((s0,), jnp.int32),
            pltpu.VMEM((nb, CHUNK, sw), jnp.float32),
            pltpu.VMEM((nb, CHUNK, dw), jnp.float32),
            pltpu.VMEM((nb, CHUNK, sw), jnp.float32),
            pltpu.SemaphoreType.DMA,
            [pltpu.SemaphoreType.DMA] * nb,
            [pltpu.SemaphoreType.DMA] * nb,
            [pltpu.SemaphoreType.DMA] * nb,
        ],
        compiler_params=pltpu.CompilerParams(use_tc_tiling_on_sc=False),
    )
    def sc_pass(s_hbm, d_hbm, ei_hbm, acc_hbm, acc_sh, src_all,
                dst_all, srows, drows, outb, sem_i, sem_g, sem_d, sem_w):
        c = lax.axis_index("c")
        s = lax.axis_index("s")
        wid = c * 16 + s

        gi1 = pltpu.async_copy(ei_hbm.at[0, pl.ds(wid * n_real, n_real)],
                               src_all.at[pl.ds(0, n_real)], sem_i)
        gi2 = pltpu.async_copy(ei_hbm.at[1, pl.ds(wid * n_real, n_real)],
                               dst_all.at[pl.ds(0, n_real)], sem_i)

        def zrow(j, _):
            def zcol(k, __):
                outb[0, j, pl.ds(k * 16, 16)] = jnp.zeros((16,), jnp.float32)
                return 0
            return lax.fori_loop(0, sw // 16, zcol, 0)
        lax.fori_loop(0, CHUNK, zrow, 0)

        def zhop(t, _):
            pltpu.sync_copy(outb.at[0, pl.ds(0, DR)],
                            acc_sh.at[pl.ds(s * rows_sub + t * DR, DR)])
            return 0
        lax.fori_loop(0, hops, zhop, 0)
        gi1.wait()
        gi2.wait()
        for t in range(nfill // 16):
            v = n0 + t * 16 + lax.iota(jnp.int32, 16)
            src_all[pl.ds(n_real + t * 16, 16)] = v
            dst_all[pl.ds(n_real + t * 16, 16)] = v

        def gathers(i, b):
            ix = pl.ds(i * CHUNK, CHUNK)
            return (pltpu.make_async_copy(s_hbm.at[src_all.at[ix]],
                                          srows.at[b], sem_g[b]),
                    pltpu.make_async_copy(d_hbm.at[dst_all.at[ix]],
                                          drows.at[b], sem_d[b]))

        def scatter(i, b):
            return pltpu.make_async_copy(
                outb.at[b], acc_sh.at[dst_all.at[pl.ds(i * CHUNK, CHUNK)]],
                sem_w[b])

        for b in range(nb):
            g1, g2 = gathers(b, b)
            g1.start()
            g2.start()
        plsc.subcore_barrier()

        def stage(i, b, t):
            g1, g2 = gathers(i, b)
            g1.wait()
            g2.wait()

            @pl.when(t > 0)
            def _():
                scatter(i, b).wait()
            compute_chunk(srows.at[b], drows.at[b], outb.at[b])
            scatter(i, b).start(add=True)

            @pl.when(i + nb < nct)
            def _():
                p1, p2 = gathers(i + nb, b)
                p1.start()
                p2.start()

        def round_(t, _):
            for b in range(nb):
                stage(nb * t + b, b, t)
            return 0
        lax.fori_loop(0, full, round_, 0)
        for b in range(rem):
            stage(nb * full + b, b, full)
        for b in range(nb):
            scatter(b, b).wait()
        plsc.subcore_barrier()

        def drain_hop(t, b):
            r0 = s * rows_sub + t * DR

            @pl.when(t >= nb)
            def _():
                pltpu.make_async_copy(outb.at[b, pl.ds(0, DR)],
                                      acc_hbm.at[pl.ds(0, DR)],
                                      sem_w[b]).wait()
            pltpu.sync_copy(acc_sh.at[pl.ds(r0, DR)], outb.at[b, pl.ds(0, DR)])
            pltpu.async_copy(outb.at[b, pl.ds(0, DR)],
                             acc_hbm.at[pl.ds(c * n_pad + r0, DR)],
                             sem_w[b])

        def dround(t, _):
            for b in range(nb):
                drain_hop(nb * t + b, b)
            return 0
        lax.fori_loop(0, hops // nb, dround, 0)
        for b in range(hops - nb * (hops // nb)):
            drain_hop(nb * (hops // nb) + b, b)
        for b in range(min(nb, hops)):
            pltpu.make_async_copy(outb.at[b, pl.ds(0, DR)],
                                  acc_hbm.at[pl.ds(0, DR)],
                                  sem_w[b]).wait()

    return sc_pass


def _cc_layer1(d1):
    def compute_chunk(srows, drows, outb):
        half = lax.shift_right_logical(lax.iota(jnp.int32, 16), 3)

        @plsc.parallel_loop(0, CHUNK, unroll=8)
        def body(j):
            ea = srows[j, pl.ds(d1, 16)] + drows[j, pl.ds(0, 16)]
            ex = _leaky_exp(ea)
            for v in range(d1 // 16):
                idx = half + 2 * v
                m = ex.at[idx].get(mode="promise_in_bounds")
                outb[j, pl.ds(16 * v, 16)] = srows[j, pl.ds(16 * v, 16)] * m
            outb[j, pl.ds(d1, 16)] = ex
    return compute_chunk


def _cc_layer2(n2):
    def compute_chunk(srows, drows, outb):
        @plsc.parallel_loop(0, CHUNK, unroll=8)
        def body(j):
            ea = srows[j, pl.ds(n2, 16)] + drows[j, pl.ds(0, 16)]
            ex = _leaky_exp(ea)
            outb[j, pl.ds(0, 16)] = srows[j, pl.ds(0, 16)] * ex
            outb[j, pl.ds(16, 16)] = ex
    return compute_chunk


def kernel(x, edge_index, W1, att_src1, att_dst1, b1, W2, att_src2, att_dst2,
           b2):
    N, F = x.shape
    H1, C1 = att_src1.shape[1], att_src1.shape[2]
    D1 = H1 * C1
    n_cls = W2.shape[1]
    E = edge_index.shape[1]

    n_pad = -(-N // 2048) * 2048
    sw1, dw1 = D1 + 16, 16
    sw2, dw2 = 2 * n_cls, n_cls

    x_pad = jnp.zeros((n_pad, F), jnp.float32).at[:N].set(x)
    n_real = E // NW
    nct = -(-n_real // CHUNK)
    if E % NW == 0 and n_real % 16 == 0 and nct * CHUNK - n_real <= n_pad - N:
        ei = edge_index.astype(jnp.int32)
    else:
        nct = NB * -(-E // (NW * CHUNK * NB))
        n_real = nct * CHUNK
        e_pad = n_real * NW
        fill = N + jnp.arange(e_pad - E, dtype=jnp.int32) % (n_pad - N)
        ei = jnp.concatenate(
            [edge_index.astype(jnp.int32),
             jnp.broadcast_to(fill, (2, e_pad - E))], axis=1)

    eye = jnp.eye(H1, dtype=jnp.float32)
    A_s = (att_src1[0][:, :, None] * eye[:, None, :]).reshape(D1, H1)
    A_d = (att_dst1[0][:, :, None] * eye[:, None, :]).reshape(D1, H1)
    v2s = att_src2.reshape(1, n_cls)
    v2d = att_dst2.reshape(1, n_cls)
    b1r = b1.reshape(1, D1)
    b2r = b2.reshape(1, n_cls)

    B = 2048
    grid = n_pad // B

    S1, T1 = pl.pallas_call(
        _tc1_body,
        grid=(grid,),
        in_specs=[
            pl.BlockSpec((B, F), lambda i: (i, 0)),
            pl.BlockSpec((F, D1), lambda i: (0, 0)),
            pl.BlockSpec((D1, H1), lambda i: (0, 0)),
            pl.BlockSpec((D1, H1), lambda i: (0, 0)),
        ],
        out_specs=[
            pl.BlockSpec((B, sw1), lambda i: (i, 0)),
            pl.BlockSpec((B, dw1), lambda i: (i, 0)),
        ],
        out_shape=[
            jax.ShapeDtypeStruct((n_pad, sw1), jnp.float32),
            jax.ShapeDtypeStruct((n_pad, dw1), jnp.float32),
        ],
    )(x_pad, W1, A_s, A_d)

    acc1 = _make_sc_pass(n_pad, sw1, dw1, nct, N, n_real,
                         _cc_layer1(D1))(S1, T1, ei)

    S2, T2 = pl.pallas_call(
        _tc2_body,
        grid=(grid,),
        in_specs=[
            pl.BlockSpec((B, sw1), lambda i: (i, 0)),
            pl.BlockSpec((B, dw1), lambda i: (i, 0)),
            pl.BlockSpec((B, sw1), lambda i: (i, 0)),
            pl.BlockSpec((B, sw1), lambda i, g=grid: (i + g, 0)),
            pl.BlockSpec((1, D1), lambda i: (0, 0)),
            pl.BlockSpec((D1, n_cls), lambda i: (0, 0)),
            pl.BlockSpec((1, n_cls), lambda i: (0, 0)),
            pl.BlockSpec((1, n_cls), lambda i: (0, 0)),
        ],
        out_specs=[
            pl.BlockSpec((B, sw2), lambda i: (i, 0)),
            pl.BlockSpec((B, dw2), lambda i: (i, 0)),
        ],
        out_shape=[
            jax.ShapeDtypeStruct((n_pad, sw2), jnp.float32),
            jax.ShapeDtypeStruct((n_pad, dw2), jnp.float32),
        ],
    )(S1, T1, acc1, acc1, b1r, W2, v2s, v2d)

    acc2 = _make_sc_pass(n_pad, sw2, dw2, nct, N, n_real,
                         _cc_layer2(n_cls))(S2, T2, ei)

    out = pl.pallas_call(
        _tc3_body,
        grid=(grid,),
        in_specs=[
            pl.BlockSpec((B, sw2), lambda i: (i, 0)),
            pl.BlockSpec((B, dw2), lambda i: (i, 0)),
            pl.BlockSpec((B, sw2), lambda i: (i, 0)),
            pl.BlockSpec((B, sw2), lambda i, g=grid: (i + g, 0)),
            pl.BlockSpec((1, n_cls), lambda i: (0, 0)),
        ],
        out_specs=pl.BlockSpec((B, n_cls), lambda i: (i, 0)),
        out_shape=jax.ShapeDtypeStruct((n_pad, n_cls), jnp.float32),
    )(S2, T2, acc2, acc2, b2r)

    return out[:N]


# speedup vs baseline: 1.1599x; 1.0792x over previous
import functools

import jax
import jax.numpy as jnp
from jax import lax
from jax.experimental import pallas as pl
from jax.experimental.pallas import tpu as pltpu
from jax.experimental.pallas import tpu_sc as plsc

NEG = 0.2
EPS = 1e-16
CHUNK = 96
NW = 32
DR = 80
NB = 3


def _leaky_exp(e):
    return jnp.exp(jnp.maximum(e, NEG * e))


def _tc1_body(x_ref, w_ref, as_ref, ad_ref, s_ref, d_ref):
    d1 = w_ref.shape[1]
    h1 = as_ref.shape[1]
    h = jnp.dot(x_ref[...], w_ref[...], preferred_element_type=jnp.float32)
    a_s = jnp.dot(h, as_ref[...], preferred_element_type=jnp.float32)
    a_d = jnp.dot(h, ad_ref[...], preferred_element_type=jnp.float32)
    s_ref[:, 0:d1] = h.astype(jnp.bfloat16)
    s_ref[:, d1:d1 + h1] = a_s.astype(jnp.bfloat16)
    s_ref[:, d1 + h1:d1 + 2 * h1] = jnp.zeros_like(a_s, jnp.bfloat16)
    d_ref[:, 0:h1] = a_d.astype(jnp.bfloat16)
    d_ref[:, h1:2 * h1] = jnp.zeros_like(a_d, jnp.bfloat16)


def _tc2_body(s1_ref, d1_ref, a0_ref, a1_ref, b1_ref, w2_ref, v2s_ref,
              v2d_ref, s2_ref, d2_ref):
    d1 = w2_ref.shape[0]
    h1 = d1_ref.shape[1] // 2
    c1 = d1 // h1
    n2 = w2_ref.shape[1]
    h = s1_ref[:, 0:d1].astype(jnp.float32)
    exs = _leaky_exp(s1_ref[:, d1:d1 + h1].astype(jnp.float32)
                     + d1_ref[:, 0:h1].astype(jnp.float32))
    num = a0_ref[:, 0:d1] + a1_ref[:, 0:d1]
    den = a0_ref[:, d1:d1 + h1] + a1_ref[:, d1:d1 + h1] + exs
    r = (lax.broadcasted_iota(jnp.int32, (h1, d1), 1) // c1
         == lax.broadcasted_iota(jnp.int32, (h1, d1), 0)).astype(jnp.float32)
    num = num + jnp.dot(exs, r, preferred_element_type=jnp.float32) * h
    den64 = jnp.dot(den, r, preferred_element_type=jnp.float32)
    out1 = num / (den64 + EPS) + b1_ref[...]
    hh = jnp.where(out1 > 0, out1, jnp.exp(out1) - 1.0)
    h2 = jnp.dot(hh, w2_ref[...], preferred_element_type=jnp.float32)
    a2s = jnp.sum(h2 * v2s_ref[...], axis=1, keepdims=True)
    a2d = jnp.sum(h2 * v2d_ref[...], axis=1, keepdims=True)
    s2_ref[:, 0:n2] = h2.astype(jnp.bfloat16)
    s2_ref[:, n2:2 * n2] = jnp.broadcast_to(a2s, h2.shape).astype(jnp.bfloat16)
    d2_ref[...] = jnp.broadcast_to(a2d, h2.shape).astype(jnp.bfloat16)


def _tc3_body(s2_ref, d2_ref, a0_ref, a1_ref, b2_ref, o_ref):
    n2 = o_ref.shape[1]
    h2 = s2_ref[:, 0:n2].astype(jnp.float32)
    exs = _leaky_exp(s2_ref[:, n2:2 * n2].astype(jnp.float32)
                     + d2_ref[...].astype(jnp.float32))
    num = a0_ref[:, 0:n2] + a1_ref[:, 0:n2] + exs * h2
    den = a0_ref[:, n2:2 * n2] + a1_ref[:, n2:2 * n2] + exs
    out = num / (den + EPS) + b2_ref[...]
    m = jnp.max(out, axis=1, keepdims=True)
    lse = m + jnp.log(jnp.sum(jnp.exp(out - m), axis=1, keepdims=True))
    o_ref[...] = out - lse


def _make_sc_pass(n_pad, sw, dw, nct, n0, n_real, compute_chunk):
    rows_sub = n_pad // 16
    hops = rows_sub // DR
    mesh = plsc.VectorSubcoreMesh(core_axis_name="c", subcore_axis_name="s",
                                  num_cores=2, num_subcores=16)

    s0 = nct * CHUNK
    nfill = s0 - n_real
    nb = NB
    full = nct // nb
    rem = nct - nb * full

    @functools.partial(
        pl.kernel,
        out_type=jax.ShapeDtypeStruct((2 * n_pad, sw), jnp.float32),
        mesh=mesh,
        scratch_types=[
            pltpu.VMEM_SHARED((n_pad, sw), jnp.float32),
            pltpu.VMEM((s0,), jnp.int32),
            pltpu.VMEM((s0,), jnp.int32),
            pltpu.VMEM((nb, CHUNK, sw), jnp.bfloat16),
            pltpu.VMEM((nb, CHUNK, dw), jnp.bfloat16),
            pltpu.VMEM((nb, CHUNK, sw), jnp.float32),
            pltpu.SemaphoreType.DMA,
            [pltpu.SemaphoreType.DMA] * nb,
            [pltpu.SemaphoreType.DMA] * nb,
            [pltpu.SemaphoreType.DMA] * nb,
        ],
        compiler_params=pltpu.CompilerParams(use_tc_tiling_on_sc=False),
    )
    def sc_pass(s_hbm, d_hbm, ei_hbm, acc_hbm, acc_sh, src_all,
                dst_all, srows, drows, outb, sem_i, sem_g, sem_d, sem_w):
        c = lax.axis_index("c")
        s = lax.axis_index("s")
        wid = c * 16 + s

        gi1 = pltpu.async_copy(ei_hbm.at[0, pl.ds(wid * n_real, n_real)],
                               src_all.at[pl.ds(0, n_real)], sem_i)
        gi2 = pltpu.async_copy(ei_hbm.at[1, pl.ds(wid * n_real, n_real)],
                               dst_all.at[pl.ds(0, n_real)], sem_i)

        def zrow(j, _):
            def zcol(k, __):
                outb[0, j, pl.ds(k * 16, 16)] = jnp.zeros((16,), jnp.float32)
                return 0
            return lax.fori_loop(0, sw // 16, zcol, 0)
        lax.fori_loop(0, CHUNK, zrow, 0)

        def zhop(t, _):
            pltpu.sync_copy(outb.at[0, pl.ds(0, DR)],
                            acc_sh.at[pl.ds(s * rows_sub + t * DR, DR)])
            return 0
        lax.fori_loop(0, hops, zhop, 0)
        gi1.wait()
        gi2.wait()
        for t in range(nfill // 16):
            v = n0 + t * 16 + lax.iota(jnp.int32, 16)
            src_all[pl.ds(n_real + t * 16, 16)] = v
            dst_all[pl.ds(n_real + t * 16, 16)] = v

        def gathers(i, b):
            ix = pl.ds(i * CHUNK, CHUNK)
            return (pltpu.make_async_copy(s_hbm.at[src_all.at[ix]],
                                          srows.at[b], sem_g[b]),
                    pltpu.make_async_copy(d_hbm.at[dst_all.at[ix]],
                                          drows.at[b], sem_d[b]))

        def scatter(i, b):
            return pltpu.make_async_copy(
                outb.at[b], acc_sh.at[dst_all.at[pl.ds(i * CHUNK, CHUNK)]],
                sem_w[b])

        for b in range(nb):
            g1, g2 = gathers(b, b)
            g1.start()
            g2.start()
        plsc.subcore_barrier()

        def stage(i, b, t):
            g1, g2 = gathers(i, b)
            g1.wait()
            g2.wait()

            @pl.when(t > 0)
            def _():
                scatter(i, b).wait()
            compute_chunk(srows.at[b], drows.at[b], outb.at[b])
            scatter(i, b).start(add=True)

            @pl.when(i + nb < nct)
            def _():
                p1, p2 = gathers(i + nb, b)
                p1.start()
                p2.start()

        def round_(t, _):
            for b in range(nb):
                stage(nb * t + b, b, t)
            return 0
        lax.fori_loop(0, full, round_, 0)
        for b in range(rem):
            stage(nb * full + b, b, full)
        for b in range(nb):
            scatter(b, b).wait()
        plsc.subcore_barrier()

        def drain_hop(t, b):
            r0 = s * rows_sub + t * DR

            @pl.when(t >= nb)
            def _():
                pltpu.make_async_copy(outb.at[b, pl.ds(0, DR)],
                                      acc_hbm.at[pl.ds(0, DR)],
                                      sem_w[b]).wait()
            pltpu.sync_copy(acc_sh.at[pl.ds(r0, DR)], outb.at[b, pl.ds(0, DR)])
            pltpu.async_copy(outb.at[b, pl.ds(0, DR)],
                             acc_hbm.at[pl.ds(c * n_pad + r0, DR)],
                             sem_w[b])

        def dround(t, _):
            for b in range(nb):
                drain_hop(nb * t + b, b)
            return 0
        lax.fori_loop(0, hops // nb, dround, 0)
        for b in range(hops - nb * (hops // nb)):
            drain_hop(nb * (hops // nb) + b, b)
        for b in range(min(nb, hops)):
            pltpu.make_async_copy(outb.at[b, pl.ds(0, DR)],
                                  acc_hbm.at[pl.ds(0, DR)],
                                  sem_w[b]).wait()

    return sc_pass


def _cc_layer1(d1):
    def compute_chunk(srows, drows, outb):
        half = lax.shift_right_logical(lax.iota(jnp.int32, 16), 3)

        @plsc.parallel_loop(0, CHUNK, unroll=8)
        def body(j):
            ea = (srows[j, pl.ds(d1, 16)].astype(jnp.float32)
                  + drows[j, pl.ds(0, 16)].astype(jnp.float32))
            ex = _leaky_exp(ea)
            for v in range(d1 // 16):
                idx = half + 2 * v
                m = ex.at[idx].get(mode="promise_in_bounds")
                outb[j, pl.ds(16 * v, 16)] = (
                    srows[j, pl.ds(16 * v, 16)].astype(jnp.float32) * m)
            outb[j, pl.ds(d1, 16)] = ex
    return compute_chunk


def _cc_layer2(n2):
    def compute_chunk(srows, drows, outb):
        @plsc.parallel_loop(0, CHUNK, unroll=8)
        def body(j):
            ea = (srows[j, pl.ds(n2, 16)].astype(jnp.float32)
                  + drows[j, pl.ds(0, 16)].astype(jnp.float32))
            ex = _leaky_exp(ea)
            outb[j, pl.ds(0, 16)] = (
                srows[j, pl.ds(0, 16)].astype(jnp.float32) * ex)
            outb[j, pl.ds(16, 16)] = ex
    return compute_chunk


def kernel(x, edge_index, W1, att_src1, att_dst1, b1, W2, att_src2, att_dst2,
           b2):
    N, F = x.shape
    H1, C1 = att_src1.shape[1], att_src1.shape[2]
    D1 = H1 * C1
    n_cls = W2.shape[1]
    E = edge_index.shape[1]

    n_pad = -(-N // 2048) * 2048
    sw1, dw1 = D1 + 16, 16
    sw2, dw2 = 2 * n_cls, n_cls

    x_pad = jnp.zeros((n_pad, F), jnp.float32).at[:N].set(x)
    n_real = E // NW
    nct = -(-n_real // CHUNK)
    if E % NW == 0 and n_real % 16 == 0 and nct * CHUNK - n_real <= n_pad - N:
        ei = edge_index.astype(jnp.int32)
    else:
        nct = NB * -(-E // (NW * CHUNK * NB))
        n_real = nct * CHUNK
        e_pad = n_real * NW
        fill = N + jnp.arange(e_pad - E, dtype=jnp.int32) % (n_pad - N)
        ei = jnp.concatenate(
            [edge_index.astype(jnp.int32),
             jnp.broadcast_to(fill, (2, e_pad - E))], axis=1)

    eye = jnp.eye(H1, dtype=jnp.float32)
    A_s = (att_src1[0][:, :, None] * eye[:, None, :]).reshape(D1, H1)
    A_d = (att_dst1[0][:, :, None] * eye[:, None, :]).reshape(D1, H1)
    v2s = att_src2.reshape(1, n_cls)
    v2d = att_dst2.reshape(1, n_cls)
    b1r = b1.reshape(1, D1)
    b2r = b2.reshape(1, n_cls)

    B = 2048
    grid = n_pad // B

    S1, T1 = pl.pallas_call(
        _tc1_body,
        grid=(grid,),
        in_specs=[
            pl.BlockSpec((B, F), lambda i: (i, 0)),
            pl.BlockSpec((F, D1), lambda i: (0, 0)),
            pl.BlockSpec((D1, H1), lambda i: (0, 0)),
            pl.BlockSpec((D1, H1), lambda i: (0, 0)),
        ],
        out_specs=[
            pl.BlockSpec((B, sw1), lambda i: (i, 0)),
            pl.BlockSpec((B, dw1), lambda i: (i, 0)),
        ],
        out_shape=[
            jax.ShapeDtypeStruct((n_pad, sw1), jnp.bfloat16),
            jax.ShapeDtypeStruct((n_pad, dw1), jnp.bfloat16),
        ],
    )(x_pad, W1, A_s, A_d)

    acc1 = _make_sc_pass(n_pad, sw1, dw1, nct, N, n_real,
                         _cc_layer1(D1))(S1, T1, ei)

    S2, T2 = pl.pallas_call(
        _tc2_body,
        grid=(grid,),
        in_specs=[
            pl.BlockSpec((B, sw1), lambda i: (i, 0)),
            pl.BlockSpec((B, dw1), lambda i: (i, 0)),
            pl.BlockSpec((B, sw1), lambda i: (i, 0)),
            pl.BlockSpec((B, sw1), lambda i, g=grid: (i + g, 0)),
            pl.BlockSpec((1, D1), lambda i: (0, 0)),
            pl.BlockSpec((D1, n_cls), lambda i: (0, 0)),
            pl.BlockSpec((1, n_cls), lambda i: (0, 0)),
            pl.BlockSpec((1, n_cls), lambda i: (0, 0)),
        ],
        out_specs=[
            pl.BlockSpec((B, sw2), lambda i: (i, 0)),
            pl.BlockSpec((B, dw2), lambda i: (i, 0)),
        ],
        out_shape=[
            jax.ShapeDtypeStruct((n_pad, sw2), jnp.bfloat16),
            jax.ShapeDtypeStruct((n_pad, dw2), jnp.bfloat16),
        ],
    )(S1, T1, acc1, acc1, b1r, W2, v2s, v2d)

    acc2 = _make_sc_pass(n_pad, sw2, dw2, nct, N, n_real,
                         _cc_layer2(n_cls))(S2, T2, ei)

    out = pl.pallas_call(
        _tc3_body,
        grid=(grid,),
        in_specs=[
            pl.BlockSpec((B, sw2), lambda i: (i, 0)),
            pl.BlockSpec((B, dw2), lambda i: (i, 0)),
            pl.BlockSpec((B, sw2), lambda i: (i, 0)),
            pl.BlockSpec((B, sw2), lambda i, g=grid: (i + g, 0)),
            pl.BlockSpec((1, n_cls), lambda i: (0, 0)),
        ],
        out_specs=pl.BlockSpec((B, n_cls), lambda i: (i, 0)),
        out_shape=jax.ShapeDtypeStruct((n_pad, n_cls), jnp.float32),
    )(S2, T2, acc2, acc2, b2r)

    return out[:N]


# speedup vs baseline: 1.1763x; 1.0142x over previous
import functools

import jax
import jax.numpy as jnp
from jax import lax
from jax.experimental import pallas as pl
from jax.experimental.pallas import tpu as pltpu
from jax.experimental.pallas import tpu_sc as plsc

NEG = 0.2
EPS = 1e-16
CHUNK = 112
NW = 32
DR = 80
NB = 3


def _leaky_exp(e):
    return jnp.exp(jnp.maximum(e, NEG * e))


def _tc1_body(x_ref, w_ref, as_ref, ad_ref, s_ref, d_ref):
    d1 = w_ref.shape[1]
    h1 = as_ref.shape[1]
    h = jnp.dot(x_ref[...], w_ref[...], preferred_element_type=jnp.float32)
    a_s = jnp.dot(h, as_ref[...], preferred_element_type=jnp.float32)
    a_d = jnp.dot(h, ad_ref[...], preferred_element_type=jnp.float32)
    s_ref[:, 0:d1] = h.astype(jnp.bfloat16)
    s_ref[:, d1:d1 + h1] = a_s.astype(jnp.bfloat16)
    s_ref[:, d1 + h1:d1 + 2 * h1] = jnp.zeros_like(a_s, jnp.bfloat16)
    d_ref[:, 0:h1] = a_d.astype(jnp.bfloat16)
    d_ref[:, h1:2 * h1] = jnp.zeros_like(a_d, jnp.bfloat16)


def _tc2_body(s1_ref, d1_ref, a0_ref, a1_ref, b1_ref, w2_ref, v2s_ref,
              v2d_ref, s2_ref, d2_ref):
    d1 = w2_ref.shape[0]
    h1 = d1_ref.shape[1] // 2
    c1 = d1 // h1
    n2 = w2_ref.shape[1]
    h = s1_ref[:, 0:d1].astype(jnp.float32)
    exs = _leaky_exp(s1_ref[:, d1:d1 + h1].astype(jnp.float32)
                     + d1_ref[:, 0:h1].astype(jnp.float32))
    num = a0_ref[:, 0:d1] + a1_ref[:, 0:d1]
    den = a0_ref[:, d1:d1 + h1] + a1_ref[:, d1:d1 + h1] + exs
    r = (lax.broadcasted_iota(jnp.int32, (h1, d1), 1) // c1
         == lax.broadcasted_iota(jnp.int32, (h1, d1), 0)).astype(jnp.float32)
    num = num + jnp.dot(exs, r, preferred_element_type=jnp.float32) * h
    den64 = jnp.dot(den, r, preferred_element_type=jnp.float32)
    out1 = num / (den64 + EPS) + b1_ref[...]
    hh = jnp.where(out1 > 0, out1, jnp.exp(out1) - 1.0)
    h2 = jnp.dot(hh, w2_ref[...], preferred_element_type=jnp.float32)
    a2s = jnp.sum(h2 * v2s_ref[...], axis=1, keepdims=True)
    a2d = jnp.sum(h2 * v2d_ref[...], axis=1, keepdims=True)
    s2_ref[:, 0:n2] = h2.astype(jnp.bfloat16)
    s2_ref[:, n2:2 * n2] = jnp.broadcast_to(a2s, h2.shape).astype(jnp.bfloat16)
    d2_ref[...] = jnp.broadcast_to(a2d, h2.shape).astype(jnp.bfloat16)


def _tc3_body(s2_ref, d2_ref, a0_ref, a1_ref, b2_ref, o_ref):
    n2 = o_ref.shape[1]
    h2 = s2_ref[:, 0:n2].astype(jnp.float32)
    exs = _leaky_exp(s2_ref[:, n2:2 * n2].astype(jnp.float32)
                     + d2_ref[...].astype(jnp.float32))
    num = a0_ref[:, 0:n2] + a1_ref[:, 0:n2] + exs * h2
    den = a0_ref[:, n2:2 * n2] + a1_ref[:, n2:2 * n2] + exs
    out = num / (den + EPS) + b2_ref[...]
    m = jnp.max(out, axis=1, keepdims=True)
    lse = m + jnp.log(jnp.sum(jnp.exp(out - m), axis=1, keepdims=True))
    o_ref[...] = out - lse


def _make_sc_pass(n_pad, sw, dw, nct, n0, n_real, compute_chunk):
    rows_sub = n_pad // 16
    hops = rows_sub // DR
    mesh = plsc.VectorSubcoreMesh(core_axis_name="c", subcore_axis_name="s",
                                  num_cores=2, num_subcores=16)

    s0 = nct * CHUNK
    nfill = s0 - n_real
    nb = NB
    full = nct // nb
    rem = nct - nb * full

    @functools.partial(
        pl.kernel,
        out_type=jax.ShapeDtypeStruct((2 * n_pad, sw), jnp.float32),
        mesh=mesh,
        scratch_types=[
            pltpu.VMEM_SHARED((n_pad, sw), jnp.float32),
            pltpu.VMEM((s0,), jnp.int32),
            pltpu.VMEM((s0,), jnp.int32),
            pltpu.VMEM((nb, CHUNK, sw), jnp.bfloat16),
            pltpu.VMEM((nb, CHUNK, dw), jnp.bfloat16),
            pltpu.VMEM((nb, CHUNK, sw), jnp.float32),
            pltpu.SemaphoreType.DMA,
            [pltpu.SemaphoreType.DMA] * nb,
            [pltpu.SemaphoreType.DMA] * nb,
            [pltpu.SemaphoreType.DMA] * nb,
        ],
        compiler_params=pltpu.CompilerParams(use_tc_tiling_on_sc=False),
    )
    def sc_pass(s_hbm, d_hbm, ei_hbm, acc_hbm, acc_sh, src_all,
                dst_all, srows, drows, outb, sem_i, sem_g, sem_d, sem_w):
        c = lax.axis_index("c")
        s = lax.axis_index("s")
        wid = c * 16 + s

        gi1 = pltpu.async_copy(ei_hbm.at[0, pl.ds(wid * n_real, n_real)],
                               src_all.at[pl.ds(0, n_real)], sem_i)
        gi2 = pltpu.async_copy(ei_hbm.at[1, pl.ds(wid * n_real, n_real)],
                               dst_all.at[pl.ds(0, n_real)], sem_i)

        def zrow(j, _):
            def zcol(k, __):
                outb[0, j, pl.ds(k * 16, 16)] = jnp.zeros((16,), jnp.float32)
                return 0
            return lax.fori_loop(0, sw // 16, zcol, 0)
        lax.fori_loop(0, CHUNK, zrow, 0)

        def zhop(t, _):
            pltpu.sync_copy(outb.at[0, pl.ds(0, DR)],
                            acc_sh.at[pl.ds(s * rows_sub + t * DR, DR)])
            return 0
        lax.fori_loop(0, hops, zhop, 0)
        gi1.wait()
        gi2.wait()
        for t in range(nfill // 16):
            v = n0 + t * 16 + lax.iota(jnp.int32, 16)
            src_all[pl.ds(n_real + t * 16, 16)] = v
            dst_all[pl.ds(n_real + t * 16, 16)] = v

        def gathers(i, b):
            ix = pl.ds(i * CHUNK, CHUNK)
            return (pltpu.make_async_copy(s_hbm.at[src_all.at[ix]],
                                          srows.at[b], sem_g[b]),
                    pltpu.make_async_copy(d_hbm.at[dst_all.at[ix]],
                                          drows.at[b], sem_d[b]))

        def scatter(i, b):
            return pltpu.make_async_copy(
                outb.at[b], acc_sh.at[dst_all.at[pl.ds(i * CHUNK, CHUNK)]],
                sem_w[b])

        for b in range(nb):
            g1, g2 = gathers(b, b)
            g1.start()
            g2.start()
        plsc.subcore_barrier()

        def stage(i, b, t):
            g1, g2 = gathers(i, b)
            g1.wait()
            g2.wait()

            @pl.when(t > 0)
            def _():
                scatter(i, b).wait()
            compute_chunk(srows.at[b], drows.at[b], outb.at[b])
            scatter(i, b).start(add=True)

            @pl.when(i + nb < nct)
            def _():
                p1, p2 = gathers(i + nb, b)
                p1.start()
                p2.start()

        def round_(t, _):
            for b in range(nb):
                stage(nb * t + b, b, t)
            return 0
        lax.fori_loop(0, full, round_, 0)
        for b in range(rem):
            stage(nb * full + b, b, full)
        for b in range(nb):
            scatter(b, b).wait()
        plsc.subcore_barrier()

        def drain_hop(t, b):
            r0 = s * rows_sub + t * DR

            @pl.when(t >= nb)
            def _():
                pltpu.make_async_copy(outb.at[b, pl.ds(0, DR)],
                                      acc_hbm.at[pl.ds(0, DR)],
                                      sem_w[b]).wait()
            pltpu.sync_copy(acc_sh.at[pl.ds(r0, DR)], outb.at[b, pl.ds(0, DR)])
            pltpu.async_copy(outb.at[b, pl.ds(0, DR)],
                             acc_hbm.at[pl.ds(c * n_pad + r0, DR)],
                             sem_w[b])

        def dround(t, _):
            for b in range(nb):
                drain_hop(nb * t + b, b)
            return 0
        lax.fori_loop(0, hops // nb, dround, 0)
        for b in range(hops - nb * (hops // nb)):
            drain_hop(nb * (hops // nb) + b, b)
        for b in range(min(nb, hops)):
            pltpu.make_async_copy(outb.at[b, pl.ds(0, DR)],
                                  acc_hbm.at[pl.ds(0, DR)],
                                  sem_w[b]).wait()

    return sc_pass


def _cc_layer1(d1):
    def compute_chunk(srows, drows, outb):
        half = lax.shift_right_logical(lax.iota(jnp.int32, 16), 3)

        @plsc.parallel_loop(0, CHUNK, unroll=8)
        def body(j):
            ea = (srows[j, pl.ds(d1, 16)].astype(jnp.float32)
                  + drows[j, pl.ds(0, 16)].astype(jnp.float32))
            ex = _leaky_exp(ea)
            for v in range(d1 // 16):
                idx = half + 2 * v
                m = ex.at[idx].get(mode="promise_in_bounds")
                outb[j, pl.ds(16 * v, 16)] = (
                    srows[j, pl.ds(16 * v, 16)].astype(jnp.float32) * m)
            outb[j, pl.ds(d1, 16)] = ex
    return compute_chunk


def _cc_layer2(n2):
    def compute_chunk(srows, drows, outb):
        @plsc.parallel_loop(0, CHUNK, unroll=8)
        def body(j):
            ea = (srows[j, pl.ds(n2, 16)].astype(jnp.float32)
                  + drows[j, pl.ds(0, 16)].astype(jnp.float32))
            ex = _leaky_exp(ea)
            outb[j, pl.ds(0, 16)] = (
                srows[j, pl.ds(0, 16)].astype(jnp.float32) * ex)
            outb[j, pl.ds(16, 16)] = ex
    return compute_chunk


def kernel(x, edge_index, W1, att_src1, att_dst1, b1, W2, att_src2, att_dst2,
           b2):
    N, F = x.shape
    H1, C1 = att_src1.shape[1], att_src1.shape[2]
    D1 = H1 * C1
    n_cls = W2.shape[1]
    E = edge_index.shape[1]

    n_pad = -(-N // 2048) * 2048
    sw1, dw1 = D1 + 16, 16
    sw2, dw2 = 2 * n_cls, n_cls

    x_pad = jnp.zeros((n_pad, F), jnp.float32).at[:N].set(x)
    n_real = E // NW
    nct = -(-n_real // CHUNK)
    if E % NW == 0 and n_real % 16 == 0 and nct * CHUNK - n_real <= n_pad - N:
        ei = edge_index.astype(jnp.int32)
    else:
        nct = NB * -(-E // (NW * CHUNK * NB))
        n_real = nct * CHUNK
        e_pad = n_real * NW
        fill = N + jnp.arange(e_pad - E, dtype=jnp.int32) % (n_pad - N)
        ei = jnp.concatenate(
            [edge_index.astype(jnp.int32),
             jnp.broadcast_to(fill, (2, e_pad - E))], axis=1)

    eye = jnp.eye(H1, dtype=jnp.float32)
    A_s = (att_src1[0][:, :, None] * eye[:, None, :]).reshape(D1, H1)
    A_d = (att_dst1[0][:, :, None] * eye[:, None, :]).reshape(D1, H1)
    v2s = att_src2.reshape(1, n_cls)
    v2d = att_dst2.reshape(1, n_cls)
    b1r = b1.reshape(1, D1)
    b2r = b2.reshape(1, n_cls)

    B = 2048
    grid = n_pad // B

    S1, T1 = pl.pallas_call(
        _tc1_body,
        grid=(grid,),
        in_specs=[
            pl.BlockSpec((B, F), lambda i: (i, 0)),
            pl.BlockSpec((F, D1), lambda i: (0, 0)),
            pl.BlockSpec((D1, H1), lambda i: (0, 0)),
            pl.BlockSpec((D1, H1), lambda i: (0, 0)),
        ],
        out_specs=[
            pl.BlockSpec((B, sw1), lambda i: (i, 0)),
            pl.BlockSpec((B, dw1), lambda i: (i, 0)),
        ],
        out_shape=[
            jax.ShapeDtypeStruct((n_pad, sw1), jnp.bfloat16),
            jax.ShapeDtypeStruct((n_pad, dw1), jnp.bfloat16),
        ],
    )(x_pad, W1, A_s, A_d)

    acc1 = _make_sc_pass(n_pad, sw1, dw1, nct, N, n_real,
                         _cc_layer1(D1))(S1, T1, ei)

    S2, T2 = pl.pallas_call(
        _tc2_body,
        grid=(grid,),
        in_specs=[
            pl.BlockSpec((B, sw1), lambda i: (i, 0)),
            pl.BlockSpec((B, dw1), lambda i: (i, 0)),
            pl.BlockSpec((B, sw1), lambda i: (i, 0)),
            pl.BlockSpec((B, sw1), lambda i, g=grid: (i + g, 0)),
            pl.BlockSpec((1, D1), lambda i: (0, 0)),
            pl.BlockSpec((D1, n_cls), lambda i: (0, 0)),
            pl.BlockSpec((1, n_cls), lambda i: (0, 0)),
            pl.BlockSpec((1, n_cls), lambda i: (0, 0)),
        ],
        out_specs=[
            pl.BlockSpec((B, sw2), lambda i: (i, 0)),
            pl.BlockSpec((B, dw2), lambda i: (i, 0)),
        ],
        out_shape=[
            jax.ShapeDtypeStruct((n_pad, sw2), jnp.bfloat16),
            jax.ShapeDtypeStruct((n_pad, dw2), jnp.bfloat16),
        ],
    )(S1, T1, acc1, acc1, b1r, W2, v2s, v2d)

    acc2 = _make_sc_pass(n_pad, sw2, dw2, nct, N, n_real,
                         _cc_layer2(n_cls))(S2, T2, ei)

    out = pl.pallas_call(
        _tc3_body,
        grid=(grid,),
        in_specs=[
            pl.BlockSpec((B, sw2), lambda i: (i, 0)),
            pl.BlockSpec((B, dw2), lambda i: (i, 0)),
            pl.BlockSpec((B, sw2), lambda i: (i, 0)),
            pl.BlockSpec((B, sw2), lambda i, g=grid: (i + g, 0)),
            pl.BlockSpec((1, n_cls), lambda i: (0, 0)),
        ],
        out_specs=pl.BlockSpec((B, n_cls), lambda i: (i, 0)),
        out_shape=jax.ShapeDtypeStruct((n_pad, n_cls), jnp.float32),
    )(S2, T2, acc2, acc2, b2r)

    return out[:N]


# speedup vs baseline: 1.1856x; 1.0079x over previous
import functools

import jax
import jax.numpy as jnp
from jax import lax
from jax.experimental import pallas as pl
from jax.experimental.pallas import tpu as pltpu
from jax.experimental.pallas import tpu_sc as plsc

NEG = 0.2
EPS = 1e-16
CHUNK = 128
NW = 32
DR = 80
NB = 3


def _leaky_exp(e):
    return jnp.exp(jnp.maximum(e, NEG * e))


def _tc1_body(x_ref, w_ref, as_ref, ad_ref, s_ref, d_ref):
    d1 = w_ref.shape[1]
    h1 = as_ref.shape[1]
    h = jnp.dot(x_ref[...], w_ref[...], preferred_element_type=jnp.float32)
    a_s = jnp.dot(h, as_ref[...], preferred_element_type=jnp.float32)
    a_d = jnp.dot(h, ad_ref[...], preferred_element_type=jnp.float32)
    s_ref[:, 0:d1] = h.astype(jnp.bfloat16)
    s_ref[:, d1:d1 + h1] = a_s.astype(jnp.bfloat16)
    s_ref[:, d1 + h1:d1 + 2 * h1] = jnp.zeros_like(a_s, jnp.bfloat16)
    d_ref[:, 0:h1] = a_d.astype(jnp.bfloat16)
    d_ref[:, h1:2 * h1] = jnp.zeros_like(a_d, jnp.bfloat16)


def _tc2_body(s1_ref, d1_ref, a0_ref, a1_ref, b1_ref, w2_ref, v2s_ref,
              v2d_ref, s2_ref, d2_ref):
    d1 = w2_ref.shape[0]
    h1 = d1_ref.shape[1] // 2
    c1 = d1 // h1
    n2 = w2_ref.shape[1]
    h = s1_ref[:, 0:d1].astype(jnp.float32)
    exs = _leaky_exp(s1_ref[:, d1:d1 + h1].astype(jnp.float32)
                     + d1_ref[:, 0:h1].astype(jnp.float32))
    num = a0_ref[:, 0:d1] + a1_ref[:, 0:d1]
    den = a0_ref[:, d1:d1 + h1] + a1_ref[:, d1:d1 + h1] + exs
    r = (lax.broadcasted_iota(jnp.int32, (h1, d1), 1) // c1
         == lax.broadcasted_iota(jnp.int32, (h1, d1), 0)).astype(jnp.float32)
    num = num + jnp.dot(exs, r, preferred_element_type=jnp.float32) * h
    den64 = jnp.dot(den, r, preferred_element_type=jnp.float32)
    out1 = num / (den64 + EPS) + b1_ref[...]
    hh = jnp.where(out1 > 0, out1, jnp.exp(out1) - 1.0)
    h2 = jnp.dot(hh, w2_ref[...], preferred_element_type=jnp.float32)
    a2s = jnp.sum(h2 * v2s_ref[...], axis=1, keepdims=True)
    a2d = jnp.sum(h2 * v2d_ref[...], axis=1, keepdims=True)
    s2_ref[:, 0:n2] = h2.astype(jnp.bfloat16)
    s2_ref[:, n2:2 * n2] = jnp.broadcast_to(a2s, h2.shape).astype(jnp.bfloat16)
    d2_ref[...] = jnp.broadcast_to(a2d, h2.shape).astype(jnp.bfloat16)


def _tc3_body(s2_ref, d2_ref, a0_ref, a1_ref, b2_ref, o_ref):
    n2 = o_ref.shape[1]
    h2 = s2_ref[:, 0:n2].astype(jnp.float32)
    exs = _leaky_exp(s2_ref[:, n2:2 * n2].astype(jnp.float32)
                     + d2_ref[...].astype(jnp.float32))
    num = a0_ref[:, 0:n2] + a1_ref[:, 0:n2] + exs * h2
    den = a0_ref[:, n2:2 * n2] + a1_ref[:, n2:2 * n2] + exs
    out = num / (den + EPS) + b2_ref[...]
    m = jnp.max(out, axis=1, keepdims=True)
    lse = m + jnp.log(jnp.sum(jnp.exp(out - m), axis=1, keepdims=True))
    o_ref[...] = out - lse


def _make_sc_pass(n_pad, sw, dw, nct, n0, n_real, compute_chunk):
    rows_sub = n_pad // 16
    hops = rows_sub // DR
    mesh = plsc.VectorSubcoreMesh(core_axis_name="c", subcore_axis_name="s",
                                  num_cores=2, num_subcores=16)

    s0 = nct * CHUNK
    nfill = s0 - n_real
    nb = NB
    full = nct // nb
    rem = nct - nb * full

    @functools.partial(
        pl.kernel,
        out_type=jax.ShapeDtypeStruct((2 * n_pad, sw), jnp.float32),
        mesh=mesh,
        scratch_types=[
            pltpu.VMEM_SHARED((n_pad, sw), jnp.float32),
            pltpu.VMEM((s0,), jnp.int32),
            pltpu.VMEM((s0,), jnp.int32),
            pltpu.VMEM((nb, CHUNK, sw), jnp.bfloat16),
            pltpu.VMEM((nb, CHUNK, dw), jnp.bfloat16),
            pltpu.VMEM((nb, CHUNK, sw), jnp.float32),
            pltpu.SemaphoreType.DMA,
            [pltpu.SemaphoreType.DMA] * nb,
            [pltpu.SemaphoreType.DMA] * nb,
            [pltpu.SemaphoreType.DMA] * nb,
        ],
        compiler_params=pltpu.CompilerParams(use_tc_tiling_on_sc=False),
    )
    def sc_pass(s_hbm, d_hbm, ei_hbm, acc_hbm, acc_sh, src_all,
                dst_all, srows, drows, outb, sem_i, sem_g, sem_d, sem_w):
        c = lax.axis_index("c")
        s = lax.axis_index("s")
        wid = c * 16 + s

        gi1 = pltpu.async_copy(ei_hbm.at[0, pl.ds(wid * n_real, n_real)],
                               src_all.at[pl.ds(0, n_real)], sem_i)
        gi2 = pltpu.async_copy(ei_hbm.at[1, pl.ds(wid * n_real, n_real)],
                               dst_all.at[pl.ds(0, n_real)], sem_i)

        def zrow(j, _):
            def zcol(k, __):
                outb[0, j, pl.ds(k * 16, 16)] = jnp.zeros((16,), jnp.float32)
                return 0
            return lax.fori_loop(0, sw // 16, zcol, 0)
        lax.fori_loop(0, CHUNK, zrow, 0)

        def zhop(t, _):
            pltpu.sync_copy(outb.at[0, pl.ds(0, DR)],
                            acc_sh.at[pl.ds(s * rows_sub + t * DR, DR)])
            return 0
        lax.fori_loop(0, hops, zhop, 0)
        gi1.wait()
        gi2.wait()
        for t in range(nfill // 16):
            v = n0 + t * 16 + lax.iota(jnp.int32, 16)
            src_all[pl.ds(n_real + t * 16, 16)] = v
            dst_all[pl.ds(n_real + t * 16, 16)] = v

        def gathers(i, b):
            ix = pl.ds(i * CHUNK, CHUNK)
            return (pltpu.make_async_copy(s_hbm.at[src_all.at[ix]],
                                          srows.at[b], sem_g[b]),
                    pltpu.make_async_copy(d_hbm.at[dst_all.at[ix]],
                                          drows.at[b], sem_d[b]))

        def scatter(i, b):
            return pltpu.make_async_copy(
                outb.at[b], acc_sh.at[dst_all.at[pl.ds(i * CHUNK, CHUNK)]],
                sem_w[b])

        for b in range(nb):
            g1, g2 = gathers(b, b)
            g1.start()
            g2.start()
        plsc.subcore_barrier()

        def stage(i, b, t):
            g1, g2 = gathers(i, b)
            g1.wait()
            g2.wait()

            @pl.when(t > 0)
            def _():
                scatter(i, b).wait()
            compute_chunk(srows.at[b], drows.at[b], outb.at[b])
            scatter(i, b).start(add=True)

            @pl.when(i + nb < nct)
            def _():
                p1, p2 = gathers(i + nb, b)
                p1.start()
                p2.start()

        def round_(t, _):
            for b in range(nb):
                stage(nb * t + b, b, t)
            return 0
        lax.fori_loop(0, full, round_, 0)
        for b in range(rem):
            stage(nb * full + b, b, full)
        for b in range(nb):
            scatter(b, b).wait()
        plsc.subcore_barrier()

        def drain_hop(t, b):
            r0 = s * rows_sub + t * DR

            @pl.when(t >= nb)
            def _():
                pltpu.make_async_copy(outb.at[b, pl.ds(0, DR)],
                                      acc_hbm.at[pl.ds(0, DR)],
                                      sem_w[b]).wait()
            pltpu.sync_copy(acc_sh.at[pl.ds(r0, DR)], outb.at[b, pl.ds(0, DR)])
            pltpu.async_copy(outb.at[b, pl.ds(0, DR)],
                             acc_hbm.at[pl.ds(c * n_pad + r0, DR)],
                             sem_w[b])

        def dround(t, _):
            for b in range(nb):
                drain_hop(nb * t + b, b)
            return 0
        lax.fori_loop(0, hops // nb, dround, 0)
        for b in range(hops - nb * (hops // nb)):
            drain_hop(nb * (hops // nb) + b, b)
        for b in range(min(nb, hops)):
            pltpu.make_async_copy(outb.at[b, pl.ds(0, DR)],
                                  acc_hbm.at[pl.ds(0, DR)],
                                  sem_w[b]).wait()

    return sc_pass


def _cc_layer1(d1):
    def compute_chunk(srows, drows, outb):
        half = lax.shift_right_logical(lax.iota(jnp.int32, 16), 3)

        @plsc.parallel_loop(0, CHUNK, unroll=8)
        def body(j):
            ea = (srows[j, pl.ds(d1, 16)].astype(jnp.float32)
                  + drows[j, pl.ds(0, 16)].astype(jnp.float32))
            ex = _leaky_exp(ea)
            for v in range(d1 // 16):
                idx = half + 2 * v
                m = ex.at[idx].get(mode="promise_in_bounds")
                outb[j, pl.ds(16 * v, 16)] = (
                    srows[j, pl.ds(16 * v, 16)].astype(jnp.float32) * m)
            outb[j, pl.ds(d1, 16)] = ex
    return compute_chunk


def _cc_layer2(n2):
    def compute_chunk(srows, drows, outb):
        @plsc.parallel_loop(0, CHUNK, unroll=8)
        def body(j):
            ea = (srows[j, pl.ds(n2, 16)].astype(jnp.float32)
                  + drows[j, pl.ds(0, 16)].astype(jnp.float32))
            ex = _leaky_exp(ea)
            outb[j, pl.ds(0, 16)] = (
                srows[j, pl.ds(0, 16)].astype(jnp.float32) * ex)
            outb[j, pl.ds(16, 16)] = ex
    return compute_chunk


def kernel(x, edge_index, W1, att_src1, att_dst1, b1, W2, att_src2, att_dst2,
           b2):
    N, F = x.shape
    H1, C1 = att_src1.shape[1], att_src1.shape[2]
    D1 = H1 * C1
    n_cls = W2.shape[1]
    E = edge_index.shape[1]

    n_pad = -(-N // 2048) * 2048
    sw1, dw1 = D1 + 16, 16
    sw2, dw2 = 2 * n_cls, n_cls

    x_pad = jnp.zeros((n_pad, F), jnp.float32).at[:N].set(x)
    n_real = E // NW
    nct = -(-n_real // CHUNK)
    if E % NW == 0 and n_real % 16 == 0 and nct * CHUNK - n_real <= n_pad - N:
        ei = edge_index.astype(jnp.int32)
    else:
        nct = NB * -(-E // (NW * CHUNK * NB))
        n_real = nct * CHUNK
        e_pad = n_real * NW
        fill = N + jnp.arange(e_pad - E, dtype=jnp.int32) % (n_pad - N)
        ei = jnp.concatenate(
            [edge_index.astype(jnp.int32),
             jnp.broadcast_to(fill, (2, e_pad - E))], axis=1)

    eye = jnp.eye(H1, dtype=jnp.float32)
    A_s = (att_src1[0][:, :, None] * eye[:, None, :]).reshape(D1, H1)
    A_d = (att_dst1[0][:, :, None] * eye[:, None, :]).reshape(D1, H1)
    v2s = att_src2.reshape(1, n_cls)
    v2d = att_dst2.reshape(1, n_cls)
    b1r = b1.reshape(1, D1)
    b2r = b2.reshape(1, n_cls)

    B = 2048
    grid = n_pad // B

    S1, T1 = pl.pallas_call(
        _tc1_body,
        grid=(grid,),
        in_specs=[
            pl.BlockSpec((B, F), lambda i: (i, 0)),
            pl.BlockSpec((F, D1), lambda i: (0, 0)),
            pl.BlockSpec((D1, H1), lambda i: (0, 0)),
            pl.BlockSpec((D1, H1), lambda i: (0, 0)),
        ],
        out_specs=[
            pl.BlockSpec((B, sw1), lambda i: (i, 0)),
            pl.BlockSpec((B, dw1), lambda i: (i, 0)),
        ],
        out_shape=[
            jax.ShapeDtypeStruct((n_pad, sw1), jnp.bfloat16),
            jax.ShapeDtypeStruct((n_pad, dw1), jnp.bfloat16),
        ],
    )(x_pad, W1, A_s, A_d)

    acc1 = _make_sc_pass(n_pad, sw1, dw1, nct, N, n_real,
                         _cc_layer1(D1))(S1, T1, ei)

    S2, T2 = pl.pallas_call(
        _tc2_body,
        grid=(grid,),
        in_specs=[
            pl.BlockSpec((B, sw1), lambda i: (i, 0)),
            pl.BlockSpec((B, dw1), lambda i: (i, 0)),
            pl.BlockSpec((B, sw1), lambda i: (i, 0)),
            pl.BlockSpec((B, sw1), lambda i, g=grid: (i + g, 0)),
            pl.BlockSpec((1, D1), lambda i: (0, 0)),
            pl.BlockSpec((D1, n_cls), lambda i: (0, 0)),
            pl.BlockSpec((1, n_cls), lambda i: (0, 0)),
            pl.BlockSpec((1, n_cls), lambda i: (0, 0)),
        ],
        out_specs=[
            pl.BlockSpec((B, sw2), lambda i: (i, 0)),
            pl.BlockSpec((B, dw2), lambda i: (i, 0)),
        ],
        out_shape=[
            jax.ShapeDtypeStruct((n_pad, sw2), jnp.bfloat16),
            jax.ShapeDtypeStruct((n_pad, dw2), jnp.bfloat16),
        ],
    )(S1, T1, acc1, acc1, b1r, W2, v2s, v2d)

    acc2 = _make_sc_pass(n_pad, sw2, dw2, nct, N, n_real,
                         _cc_layer2(n_cls))(S2, T2, ei)

    out = pl.pallas_call(
        _tc3_body,
        grid=(grid,),
        in_specs=[
            pl.BlockSpec((B, sw2), lambda i: (i, 0)),
            pl.BlockSpec((B, dw2), lambda i: (i, 0)),
            pl.BlockSpec((B, sw2), lambda i: (i, 0)),
            pl.BlockSpec((B, sw2), lambda i, g=grid: (i + g, 0)),
            pl.BlockSpec((1, n_cls), lambda i: (0, 0)),
        ],
        out_specs=pl.BlockSpec((B, n_cls), lambda i: (i, 0)),
        out_shape=jax.ShapeDtypeStruct((n_pad, n_cls), jnp.float32),
    )(S2, T2, acc2, acc2, b2r)

    return out[:N]
